# Initial kernel scaffold; baseline (speedup 1.0000x reference)
import functools

import jax
import jax.numpy as jnp
from jax import lax
from jax.experimental import pallas as pl
from jax.experimental.pallas import tpu as pltpu
from jax.experimental.pallas import tpu_sc as plsc

N = 10000
E = 320000
D = 128

NC = 2
NS = 16
NW = NC * NS
K = 125
CH = E // (NW * K)
RPT = N // NS

_HI = jax.lax.Precision.HIGHEST

_sc_mesh = plsc.VectorSubcoreMesh(
    core_axis_name="c", subcore_axis_name="s", num_cores=NC, num_subcores=NS
)


@functools.partial(
    pl.kernel,
    out_type=jax.ShapeDtypeStruct((E, D), jnp.float32),
    mesh=_sc_mesh,
    scratch_types=[
        pltpu.VMEM((K,), jnp.int32),
        pltpu.VMEM((K,), jnp.int32),
        pltpu.VMEM((K, D), jnp.float32),
        pltpu.VMEM((K, D), jnp.float32),
        pltpu.SemaphoreType.DMA,
        pltpu.SemaphoreType.DMA,
    ],
)
def _sc_gather(p_hbm, q_hbm, srcr_hbm, dstr_hbm, s_hbm,
               idx_s, idx_d, prow, qrow, sem1, sem2):
    cid = lax.axis_index("c")
    sid = lax.axis_index("s")
    wid = cid * NS + sid

    def body(j, carry):
        pltpu.sync_copy(srcr_hbm.at[wid, j], idx_s)
        pltpu.sync_copy(dstr_hbm.at[wid, j], idx_d)
        cp1 = pltpu.async_copy(p_hbm.at[idx_s], prow, sem1)
        cp2 = pltpu.async_copy(q_hbm.at[idx_d], qrow, sem2)
        cp1.wait()
        cp2.wait()

        def add_row(r, c):
            for u in range(D // 16):
                sl = pl.ds(u * 16, 16)
                prow[r, sl] = prow[r, sl] + qrow[r, sl]
            return c

        lax.fori_loop(0, K, add_row, 0)
        pltpu.sync_copy(prow, s_hbm.at[pl.ds((wid * CH + j) * K, K)])
        return carry

    lax.fori_loop(0, CH, body, 0)


@functools.partial(
    pl.kernel,
    out_type=jax.ShapeDtypeStruct((NC, N, D), jnp.float32),
    mesh=_sc_mesh,
    scratch_types=[
        pltpu.VMEM((K,), jnp.int32),
        pltpu.VMEM((K, D), jnp.float32),
        pltpu.VMEM_SHARED((N, D), jnp.float32),
        pltpu.SemaphoreType.DMA,
    ],
)
def _sc_scatter(eupd_hbm, dstr_hbm, zeros_hbm, out_hbm,
                idx_d, rows, agg_sp, sem):
    cid = lax.axis_index("c")
    sid = lax.axis_index("s")
    wid = cid * NS + sid

    pltpu.sync_copy(zeros_hbm.at[pl.ds(sid * RPT, RPT)],
                    agg_sp.at[pl.ds(sid * RPT, RPT)])
    plsc.subcore_barrier()

    def body(j, carry):
        base = (wid * CH + j) * K
        pltpu.sync_copy(dstr_hbm.at[wid, j], idx_d)
        pltpu.sync_copy(eupd_hbm.at[pl.ds(base, K)], rows)
        pltpu.sync_copy(rows, agg_sp.at[idx_d], add=True)
        return carry

    lax.fori_loop(0, CH, body, 0)
    plsc.subcore_barrier()
    pltpu.sync_copy(agg_sp.at[pl.ds(sid * RPT, RPT)],
                    out_hbm.at[cid, pl.ds(sid * RPT, RPT)])


def _proj_body(v_ref, wa_ref, wb_ref, p_ref, q_ref):
    v = v_ref[...]
    p_ref[...] = lax.dot_general(v, wa_ref[...], (((1,), (0,)), ((), ())),
                                 precision=_HI)
    q_ref[...] = lax.dot_general(v, wb_ref[...], (((1,), (0,)), ((), ())),
                                 precision=_HI)


def _edge_body(s_ref, e_ref, wc_ref, be_ref, enew_ref, eupd_ref):
    e = e_ref[...]
    acc = s_ref[...] + lax.dot_general(e, wc_ref[...], (((1,), (0,)), ((), ())),
                                       precision=_HI) + be_ref[...]
    upd = jnp.maximum(acc, 0.0)
    eupd_ref[...] = upd
    enew_ref[...] = e + upd


def _vtx_body(v_ref, a0_ref, a1_ref, wva_ref, wvb_ref, bv_ref, out_ref):
    v = v_ref[...]
    agg = a0_ref[...] + a1_ref[...]
    h = (lax.dot_general(v, wva_ref[...], (((1,), (0,)), ((), ())),
                         precision=_HI)
         + lax.dot_general(agg, wvb_ref[...], (((1,), (0,)), ((), ())),
                           precision=_HI)
         + bv_ref[...])
    out_ref[...] = v + jnp.maximum(h, 0.0)


_VB = 2000
_EB = 4000


def _proj(v, wa, wb):
    return pl.pallas_call(
        _proj_body,
        grid=(N // _VB,),
        in_specs=[
            pl.BlockSpec((_VB, D), lambda i: (i, 0)),
            pl.BlockSpec((D, D), lambda i: (0, 0)),
            pl.BlockSpec((D, D), lambda i: (0, 0)),
        ],
        out_specs=[
            pl.BlockSpec((_VB, D), lambda i: (i, 0)),
            pl.BlockSpec((_VB, D), lambda i: (i, 0)),
        ],
        out_shape=[
            jax.ShapeDtypeStruct((N, D), jnp.float32),
            jax.ShapeDtypeStruct((N, D), jnp.float32),
        ],
    )(v, wa, wb)


def _edge(s, e, wc, be2d):
    return pl.pallas_call(
        _edge_body,
        grid=(E // _EB,),
        in_specs=[
            pl.BlockSpec((_EB, D), lambda i: (i, 0)),
            pl.BlockSpec((_EB, D), lambda i: (i, 0)),
            pl.BlockSpec((D, D), lambda i: (0, 0)),
            pl.BlockSpec((1, D), lambda i: (0, 0)),
        ],
        out_specs=[
            pl.BlockSpec((_EB, D), lambda i: (i, 0)),
            pl.BlockSpec((_EB, D), lambda i: (i, 0)),
        ],
        out_shape=[
            jax.ShapeDtypeStruct((E, D), jnp.float32),
            jax.ShapeDtypeStruct((E, D), jnp.float32),
        ],
    )(s, e, wc, be2d)


def _vtx(v, a0, a1, wva, wvb, bv2d):
    return pl.pallas_call(
        _vtx_body,
        grid=(N // _VB,),
        in_specs=[
            pl.BlockSpec((_VB, D), lambda i: (i, 0)),
            pl.BlockSpec((_VB, D), lambda i: (i, 0)),
            pl.BlockSpec((_VB, D), lambda i: (i, 0)),
            pl.BlockSpec((D, D), lambda i: (0, 0)),
            pl.BlockSpec((D, D), lambda i: (0, 0)),
            pl.BlockSpec((1, D), lambda i: (0, 0)),
        ],
        out_specs=pl.BlockSpec((_VB, D), lambda i: (i, 0)),
        out_shape=jax.ShapeDtypeStruct((N, D), jnp.float32),
    )(v, a0, a1, wva, wvb, bv2d)


def kernel(vertex_features, edge_features, edge_index,
           We0, be0, Wv0, bv0, We1, be1, Wv1, bv1):
    srcr = edge_index[0].reshape(NW, CH, K)
    dstr = edge_index[1].reshape(NW, CH, K)
    zeros = jnp.zeros((N, D), jnp.float32)

    v, e = vertex_features, edge_features
    for (We, be, Wv, bv) in ((We0, be0, Wv0, bv0), (We1, be1, Wv1, bv1)):
        p, q = _proj(v, We[:D], We[D:2 * D])
        s = _sc_gather(p, q, srcr, dstr)
        e, eupd = _edge(s, e, We[2 * D:], be.reshape(1, D))
        aggp = _sc_scatter(eupd, dstr, zeros)
        v = _vtx(v, aggp[0], aggp[1], Wv[:D], Wv[D:], bv.reshape(1, D))
    return v, e


# speedup vs baseline: 3.0595x; 3.0595x over previous
import functools

import jax
import jax.numpy as jnp
from jax import lax
from jax.experimental import pallas as pl
from jax.experimental.pallas import tpu as pltpu
from jax.experimental.pallas import tpu_sc as plsc

N = 10000
E = 320000
D = 128

NC = 2
NS = 16
NW = NC * NS
K = 128
CHUNKS = E // K
CH = -(-CHUNKS // NW)
NP = 10240
RPT = NP // NS

_HI = jax.lax.Precision.HIGHEST


def _sc_gather_body(p_hbm, q_hbm, src_hbm, dst_hbm, s_hbm,
                    idx_s, idx_d, prow, qrow, sem1, sem2):
    cid = lax.axis_index("c")
    sid = lax.axis_index("s")
    wid = cid * NS + sid

    def body(j, carry):
        c = wid + j * NW

        @pl.when(c < CHUNKS)
        def _():
            base = c * K
            pltpu.sync_copy(src_hbm.at[pl.ds(base, K)], idx_s)
            pltpu.sync_copy(dst_hbm.at[pl.ds(base, K)], idx_d)
            cp1 = pltpu.async_copy(p_hbm.at[idx_s], prow, sem1)
            cp2 = pltpu.async_copy(q_hbm.at[idx_d], qrow, sem2)
            cp1.wait()
            cp2.wait()

            def add_row(r, cc):
                for u in range(D // 16):
                    sl = pl.ds(u * 16, 16)
                    prow[r, sl] = prow[r, sl] + qrow[r, sl]
                return cc

            lax.fori_loop(0, K, add_row, 0)
            pltpu.sync_copy(prow, s_hbm.at[pl.ds(base, K)])

        return carry

    lax.fori_loop(0, CH, body, 0)


def _sc_scatter_body(eupd_hbm, dst_hbm, zeros_hbm, out_hbm,
                     idx_d, rows, agg_sp, sem):
    cid = lax.axis_index("c")
    sid = lax.axis_index("s")
    wid = cid * NS + sid

    pltpu.sync_copy(zeros_hbm.at[pl.ds(sid * RPT, RPT)],
                    agg_sp.at[pl.ds(sid * RPT, RPT)])
    plsc.subcore_barrier()

    def body(j, carry):
        c = wid + j * NW

        @pl.when(c < CHUNKS)
        def _():
            base = c * K
            pltpu.sync_copy(dst_hbm.at[pl.ds(base, K)], idx_d)
            pltpu.sync_copy(eupd_hbm.at[pl.ds(base, K)], rows)
            pltpu.sync_copy(rows, agg_sp.at[idx_d], add=True)

        return carry

    lax.fori_loop(0, CH, body, 0)
    plsc.subcore_barrier()
    pltpu.sync_copy(agg_sp.at[pl.ds(sid * RPT, RPT)],
                    out_hbm.at[cid, pl.ds(sid * RPT, RPT)])


@functools.lru_cache(maxsize=None)
def _sc_kernels():
    mesh = plsc.VectorSubcoreMesh(
        core_axis_name="c", subcore_axis_name="s",
        num_cores=NC, num_subcores=NS,
    )
    gather = pl.kernel(
        _sc_gather_body,
        out_type=jax.ShapeDtypeStruct((E, D), jnp.float32),
        mesh=mesh,
        scratch_types=[
            pltpu.VMEM((K,), jnp.int32),
            pltpu.VMEM((K,), jnp.int32),
            pltpu.VMEM((K, D), jnp.float32),
            pltpu.VMEM((K, D), jnp.float32),
            pltpu.SemaphoreType.DMA,
            pltpu.SemaphoreType.DMA,
        ],
    )
    scatter = pl.kernel(
        _sc_scatter_body,
        out_type=jax.ShapeDtypeStruct((NC, NP, D), jnp.float32),
        mesh=mesh,
        scratch_types=[
            pltpu.VMEM((K,), jnp.int32),
            pltpu.VMEM((K, D), jnp.float32),
            pltpu.VMEM_SHARED((NP, D), jnp.float32),
            pltpu.SemaphoreType.DMA,
        ],
    )
    return gather, scatter


def _proj_body(v_ref, wa_ref, wb_ref, p_ref, q_ref):
    v = v_ref[...]
    p_ref[...] = lax.dot_general(v, wa_ref[...], (((1,), (0,)), ((), ())),
                                 precision=_HI)
    q_ref[...] = lax.dot_general(v, wb_ref[...], (((1,), (0,)), ((), ())),
                                 precision=_HI)


def _edge_body(s_ref, e_ref, wc_ref, be_ref, enew_ref, eupd_ref):
    e = e_ref[...]
    acc = s_ref[...] + lax.dot_general(e, wc_ref[...], (((1,), (0,)), ((), ())),
                                       precision=_HI) + be_ref[...]
    upd = jnp.maximum(acc, 0.0)
    eupd_ref[...] = upd
    enew_ref[...] = e + upd


def _vtx_body(v_ref, a0_ref, a1_ref, wva_ref, wvb_ref, bv_ref, out_ref):
    v = v_ref[...]
    agg = a0_ref[...] + a1_ref[...]
    h = (lax.dot_general(v, wva_ref[...], (((1,), (0,)), ((), ())),
                         precision=_HI)
         + lax.dot_general(agg, wvb_ref[...], (((1,), (0,)), ((), ())),
                           precision=_HI)
         + bv_ref[...])
    out_ref[...] = v + jnp.maximum(h, 0.0)


_VB = 2000
_EB = 4000


def _proj(v, wa, wb):
    return pl.pallas_call(
        _proj_body,
        grid=(N // _VB,),
        in_specs=[
            pl.BlockSpec((_VB, D), lambda i: (i, 0)),
            pl.BlockSpec((D, D), lambda i: (0, 0)),
            pl.BlockSpec((D, D), lambda i: (0, 0)),
        ],
        out_specs=[
            pl.BlockSpec((_VB, D), lambda i: (i, 0)),
            pl.BlockSpec((_VB, D), lambda i: (i, 0)),
        ],
        out_shape=[
            jax.ShapeDtypeStruct((N, D), jnp.float32),
            jax.ShapeDtypeStruct((N, D), jnp.float32),
        ],
    )(v, wa, wb)


def _edge(s, e, wc, be2d):
    return pl.pallas_call(
        _edge_body,
        grid=(E // _EB,),
        in_specs=[
            pl.BlockSpec((_EB, D), lambda i: (i, 0)),
            pl.BlockSpec((_EB, D), lambda i: (i, 0)),
            pl.BlockSpec((D, D), lambda i: (0, 0)),
            pl.BlockSpec((1, D), lambda i: (0, 0)),
        ],
        out_specs=[
            pl.BlockSpec((_EB, D), lambda i: (i, 0)),
            pl.BlockSpec((_EB, D), lambda i: (i, 0)),
        ],
        out_shape=[
            jax.ShapeDtypeStruct((E, D), jnp.float32),
            jax.ShapeDtypeStruct((E, D), jnp.float32),
        ],
    )(s, e, wc, be2d)


def _vtx(v, a0, a1, wva, wvb, bv2d):
    return pl.pallas_call(
        _vtx_body,
        grid=(N // _VB,),
        in_specs=[
            pl.BlockSpec((_VB, D), lambda i: (i, 0)),
            pl.BlockSpec((_VB, D), lambda i: (i, 0)),
            pl.BlockSpec((_VB, D), lambda i: (i, 0)),
            pl.BlockSpec((D, D), lambda i: (0, 0)),
            pl.BlockSpec((D, D), lambda i: (0, 0)),
            pl.BlockSpec((1, D), lambda i: (0, 0)),
        ],
        out_specs=pl.BlockSpec((_VB, D), lambda i: (i, 0)),
        out_shape=jax.ShapeDtypeStruct((N, D), jnp.float32),
    )(v, a0, a1, wva, wvb, bv2d)


def kernel(vertex_features, edge_features, edge_index,
           We0, be0, Wv0, bv0, We1, be1, Wv1, bv1):
    src = edge_index[0]
    dst = edge_index[1]
    zeros = jnp.zeros((NP, D), jnp.float32)

    sc_gather, sc_scatter = _sc_kernels()
    v, e = vertex_features, edge_features
    for (We, be, Wv, bv) in ((We0, be0, Wv0, bv0), (We1, be1, Wv1, bv1)):
        p, q = _proj(v, We[:D], We[D:2 * D])
        s = sc_gather(p, q, src, dst)
        e, eupd = _edge(s, e, We[2 * D:], be.reshape(1, D))
        aggp = sc_scatter(eupd, dst, zeros)
        v = _vtx(v, aggp[0], aggp[1], Wv[:D], Wv[D:], bv.reshape(1, D))
    return v, e


# speedup vs baseline: 4.7587x; 1.5553x over previous
import functools

import jax
import jax.numpy as jnp
from jax import lax
from jax.experimental import pallas as pl
from jax.experimental.pallas import tpu as pltpu
from jax.experimental.pallas import tpu_sc as plsc

N = 10000
E = 320000
D = 128

NC = 2
NS = 16
NW = NC * NS
K = 80
WE = E // NW
CH = WE // K
NB = 4
NP = 10240
RPT = NP // NS

GB = K * D * 4
IB = K * 4

_HI = jax.lax.Precision.HIGHEST


def _sc_gather_body(p_hbm, q_hbm, src_hbm, dst_hbm, s_hbm,
                    idxs, idxd, prow, qrow,
                    si0, si1, si2, si3, sg0, sg1, sg2, sg3,
                    st0, st1, st2, st3):
    sem_i = (si0, si1, si2, si3)
    sem_g = (sg0, sg1, sg2, sg3)
    sem_st = (st0, st1, st2, st3)
    cid = lax.axis_index("c")
    sid = lax.axis_index("s")
    wid = cid * NS + sid
    w0 = wid * WE

    def fire_idx(j, b):
        pltpu.async_copy(src_hbm.at[pl.ds(w0 + j * K, K)], idxs.at[b], sem_i[b])
        pltpu.async_copy(dst_hbm.at[pl.ds(w0 + j * K, K)], idxd.at[b], sem_i[b])

    def fire_gather(b):
        pltpu.async_copy(p_hbm.at[idxs.at[b]], prow.at[b], sem_g[b])
        pltpu.async_copy(q_hbm.at[idxd.at[b]], qrow.at[b], sem_g[b])

    def wait_idx(b):
        pltpu.make_async_copy(src_hbm.at[pl.ds(w0, K)], idxs.at[b], sem_i[b]).wait()
        pltpu.make_async_copy(dst_hbm.at[pl.ds(w0, K)], idxd.at[b], sem_i[b]).wait()

    def wait_gather(b):
        pltpu.make_async_copy(p_hbm.at[pl.ds(0, K)], prow.at[b], sem_g[b]).wait()
        pltpu.make_async_copy(q_hbm.at[pl.ds(0, K)], qrow.at[b], sem_g[b]).wait()

    def wait_store(b):
        pltpu.make_async_copy(prow.at[b], s_hbm.at[pl.ds(w0, K)], sem_st[b]).wait()

    fire_idx(0, 0)
    fire_idx(1, 1)
    fire_idx(2, 2)
    wait_idx(0)
    fire_gather(0)
    wait_idx(1)
    fire_gather(1)

    def iter_j(j, b):
        bn = (b + 2) % NB
        bi = (b + 3) % NB

        @pl.when(j >= 2)
        def _():
            wait_store(bn)

        @pl.when(j + 3 < CH)
        def _():
            fire_idx(j + 3, bi)

        @pl.when(j + 2 < CH)
        def _():
            wait_idx(bn)
            fire_gather(bn)

        wait_gather(b)

        def add_row(r, cc):
            for u in range(D // 16):
                sl = pl.ds(u * 16, 16)
                prow[b, r, sl] = prow[b, r, sl] + qrow[b, r, sl]
            return cc

        lax.fori_loop(0, K, add_row, 0)
        pltpu.async_copy(prow.at[b], s_hbm.at[pl.ds(w0 + j * K, K)], sem_st[b])

    def outer(t, carry):
        for u in range(NB):
            j = t * NB + u

            @pl.when(j < CH)
            def _():
                iter_j(j, u)

        return carry

    lax.fori_loop(0, -(-CH // NB), outer, 0)
    wait_store((CH - 2) % NB)
    wait_store((CH - 1) % NB)


def _sc_scatter_body(eupd_hbm, dst_hbm, zeros_hbm, out_hbm,
                     idxd, rows, agg_sp,
                     sr0, sr1, sr2, sr3, ss0, ss1, ss2, ss3):
    sem_r = (sr0, sr1, sr2, sr3)
    sem_s = (ss0, ss1, ss2, ss3)
    cid = lax.axis_index("c")
    sid = lax.axis_index("s")
    wid = cid * NS + sid
    w0 = wid * WE

    pltpu.sync_copy(zeros_hbm.at[pl.ds(sid * RPT, RPT)],
                    agg_sp.at[pl.ds(sid * RPT, RPT)])
    plsc.subcore_barrier()

    def fire(j, b):
        pltpu.async_copy(dst_hbm.at[pl.ds(w0 + j * K, K)], idxd.at[b], sem_r[b])
        pltpu.async_copy(eupd_hbm.at[pl.ds(w0 + j * K, K)], rows.at[b], sem_r[b])

    def wait_rows(b):
        pltpu.make_async_copy(dst_hbm.at[pl.ds(w0, K)], idxd.at[b], sem_r[b]).wait()
        pltpu.make_async_copy(eupd_hbm.at[pl.ds(w0, K)], rows.at[b], sem_r[b]).wait()

    def wait_scatter(b):
        pltpu.make_async_copy(rows.at[b], agg_sp.at[pl.ds(0, K)], sem_s[b]).wait()

    fire(0, 0)
    fire(1, 1)

    def iter_j(j, b):
        bn = (b + 2) % NB

        @pl.when(j >= 2)
        def _():
            wait_scatter(bn)

        @pl.when(j + 2 < CH)
        def _():
            fire(j + 2, bn)

        wait_rows(b)
        pltpu.async_copy(rows.at[b], agg_sp.at[idxd.at[b]], sem_s[b], add=True)

    def outer(t, carry):
        for u in range(NB):
            j = t * NB + u

            @pl.when(j < CH)
            def _():
                iter_j(j, u)

        return carry

    lax.fori_loop(0, -(-CH // NB), outer, 0)
    wait_scatter((CH - 2) % NB)
    wait_scatter((CH - 1) % NB)
    plsc.subcore_barrier()
    pltpu.sync_copy(agg_sp.at[pl.ds(sid * RPT, RPT)],
                    out_hbm.at[cid, pl.ds(sid * RPT, RPT)])


@functools.lru_cache(maxsize=None)
def _sc_kernels():
    mesh = plsc.VectorSubcoreMesh(
        core_axis_name="c", subcore_axis_name="s",
        num_cores=NC, num_subcores=NS,
    )
    gather = pl.kernel(
        _sc_gather_body,
        out_type=jax.ShapeDtypeStruct((E, D), jnp.float32),
        mesh=mesh,
        scratch_types=(
            [
                pltpu.VMEM((NB, K), jnp.int32),
                pltpu.VMEM((NB, K), jnp.int32),
                pltpu.VMEM((NB, K, D), jnp.float32),
                pltpu.VMEM((NB, K, D), jnp.float32),
            ]
            + [pltpu.SemaphoreType.DMA] * 12
        ),
    )
    scatter = pl.kernel(
        _sc_scatter_body,
        out_type=jax.ShapeDtypeStruct((NC, NP, D), jnp.float32),
        mesh=mesh,
        scratch_types=(
            [
                pltpu.VMEM((NB, K), jnp.int32),
                pltpu.VMEM((NB, K, D), jnp.float32),
                pltpu.VMEM_SHARED((NP, D), jnp.float32),
            ]
            + [pltpu.SemaphoreType.DMA] * 8
        ),
    )
    return gather, scatter


def _proj_body(v_ref, wa_ref, wb_ref, p_ref, q_ref):
    v = v_ref[...]
    p_ref[...] = lax.dot_general(v, wa_ref[...], (((1,), (0,)), ((), ())),
                                 precision=_HI)
    q_ref[...] = lax.dot_general(v, wb_ref[...], (((1,), (0,)), ((), ())),
                                 precision=_HI)


def _edge_body(s_ref, e_ref, wc_ref, be_ref, enew_ref, eupd_ref):
    e = e_ref[...]
    acc = s_ref[...] + lax.dot_general(e, wc_ref[...], (((1,), (0,)), ((), ())),
                                       precision=_HI) + be_ref[...]
    upd = jnp.maximum(acc, 0.0)
    eupd_ref[...] = upd
    enew_ref[...] = e + upd


def _vtx_body(v_ref, a0_ref, a1_ref, wva_ref, wvb_ref, bv_ref, out_ref):
    v = v_ref[...]
    agg = a0_ref[...] + a1_ref[...]
    h = (lax.dot_general(v, wva_ref[...], (((1,), (0,)), ((), ())),
                         precision=_HI)
         + lax.dot_general(agg, wvb_ref[...], (((1,), (0,)), ((), ())),
                           precision=_HI)
         + bv_ref[...])
    out_ref[...] = v + jnp.maximum(h, 0.0)


_VB = 2000
_EB = 4000


def _proj(v, wa, wb):
    return pl.pallas_call(
        _proj_body,
        grid=(N // _VB,),
        in_specs=[
            pl.BlockSpec((_VB, D), lambda i: (i, 0)),
            pl.BlockSpec((D, D), lambda i: (0, 0)),
            pl.BlockSpec((D, D), lambda i: (0, 0)),
        ],
        out_specs=[
            pl.BlockSpec((_VB, D), lambda i: (i, 0)),
            pl.BlockSpec((_VB, D), lambda i: (i, 0)),
        ],
        out_shape=[
            jax.ShapeDtypeStruct((N, D), jnp.float32),
            jax.ShapeDtypeStruct((N, D), jnp.float32),
        ],
    )(v, wa, wb)


def _edge(s, e, wc, be2d):
    return pl.pallas_call(
        _edge_body,
        grid=(E // _EB,),
        in_specs=[
            pl.BlockSpec((_EB, D), lambda i: (i, 0)),
            pl.BlockSpec((_EB, D), lambda i: (i, 0)),
            pl.BlockSpec((D, D), lambda i: (0, 0)),
            pl.BlockSpec((1, D), lambda i: (0, 0)),
        ],
        out_specs=[
            pl.BlockSpec((_EB, D), lambda i: (i, 0)),
            pl.BlockSpec((_EB, D), lambda i: (i, 0)),
        ],
        out_shape=[
            jax.ShapeDtypeStruct((E, D), jnp.float32),
            jax.ShapeDtypeStruct((E, D), jnp.float32),
        ],
    )(s, e, wc, be2d)


def _vtx(v, a0, a1, wva, wvb, bv2d):
    return pl.pallas_call(
        _vtx_body,
        grid=(N // _VB,),
        in_specs=[
            pl.BlockSpec((_VB, D), lambda i: (i, 0)),
            pl.BlockSpec((_VB, D), lambda i: (i, 0)),
            pl.BlockSpec((_VB, D), lambda i: (i, 0)),
            pl.BlockSpec((D, D), lambda i: (0, 0)),
            pl.BlockSpec((D, D), lambda i: (0, 0)),
            pl.BlockSpec((1, D), lambda i: (0, 0)),
        ],
        out_specs=pl.BlockSpec((_VB, D), lambda i: (i, 0)),
        out_shape=jax.ShapeDtypeStruct((N, D), jnp.float32),
    )(v, a0, a1, wva, wvb, bv2d)


def kernel(vertex_features, edge_features, edge_index,
           We0, be0, Wv0, bv0, We1, be1, Wv1, bv1):
    src = edge_index[0]
    dst = edge_index[1]
    zeros = jnp.zeros((NP, D), jnp.float32)

    sc_gather, sc_scatter = _sc_kernels()
    v, e = vertex_features, edge_features
    for (We, be, Wv, bv) in ((We0, be0, Wv0, bv0), (We1, be1, Wv1, bv1)):
        p, q = _proj(v, We[:D], We[D:2 * D])
        s = sc_gather(p, q, src, dst)
        e, eupd = _edge(s, e, We[2 * D:], be.reshape(1, D))
        aggp = sc_scatter(eupd, dst, zeros)
        v = _vtx(v, aggp[0], aggp[1], Wv[:D], Wv[D:], bv.reshape(1, D))
    return v, e


# speedup vs baseline: 5.0087x; 1.0525x over previous
import functools

import jax
import jax.numpy as jnp
from jax import lax
from jax.experimental import pallas as pl
from jax.experimental.pallas import tpu as pltpu
from jax.experimental.pallas import tpu_sc as plsc

N = 10000
E = 320000
D = 128

NC = 2
NS = 16
NW = NC * NS
K = 80
WE = E // NW
CH = WE // K
NB = 4
NP = 10240
RPT = NP // NS

DW = D // 2

GB = K * D * 4
IB = K * 4

_HI = jax.lax.Precision.HIGHEST


def _sc_gather_body(p_hbm, q_hbm, src_hbm, dst_hbm, s_hbm,
                    idxs, idxd, prow, qrow,
                    si0, si1, si2, si3, sg0, sg1, sg2, sg3,
                    st0, st1, st2, st3):
    sem_i = (si0, si1, si2, si3)
    sem_g = (sg0, sg1, sg2, sg3)
    sem_st = (st0, st1, st2, st3)
    cid = lax.axis_index("c")
    sid = lax.axis_index("s")
    wid = cid * NS + sid
    w0 = wid * WE

    def fire_idx(j, b):
        pltpu.async_copy(src_hbm.at[pl.ds(w0 + j * K, K)], idxs.at[b], sem_i[b])
        pltpu.async_copy(dst_hbm.at[pl.ds(w0 + j * K, K)], idxd.at[b], sem_i[b])

    def fire_gather(b):
        pltpu.async_copy(p_hbm.at[idxs.at[b]], prow.at[b], sem_g[b])
        pltpu.async_copy(q_hbm.at[idxd.at[b]], qrow.at[b], sem_g[b])

    def wait_idx(b):
        pltpu.make_async_copy(src_hbm.at[pl.ds(w0, K)], idxs.at[b], sem_i[b]).wait()
        pltpu.make_async_copy(dst_hbm.at[pl.ds(w0, K)], idxd.at[b], sem_i[b]).wait()

    def wait_gather(b):
        pltpu.make_async_copy(p_hbm.at[pl.ds(0, K)], prow.at[b], sem_g[b]).wait()
        pltpu.make_async_copy(q_hbm.at[pl.ds(0, K)], qrow.at[b], sem_g[b]).wait()

    def wait_store(b):
        pltpu.make_async_copy(prow.at[b], s_hbm.at[pl.ds(w0, K)], sem_st[b]).wait()

    fire_idx(0, 0)
    fire_idx(1, 1)
    fire_idx(2, 2)
    wait_idx(0)
    fire_gather(0)
    wait_idx(1)
    fire_gather(1)

    def iter_j(j, b):
        bn = (b + 2) % NB
        bi = (b + 3) % NB

        @pl.when(j >= 2)
        def _():
            wait_store(bn)

        @pl.when(j + 3 < CH)
        def _():
            fire_idx(j + 3, bi)

        @pl.when(j + 2 < CH)
        def _():
            wait_idx(bn)
            fire_gather(bn)

        wait_gather(b)

        def add_row(r, cc):
            for u in range(D // 16):
                sl = pl.ds(u * 16, 16)
                prow[b, r, sl] = prow[b, r, sl] + qrow[b, r, sl]
            return cc

        lax.fori_loop(0, K, add_row, 0)
        pltpu.async_copy(prow.at[b], s_hbm.at[pl.ds(w0 + j * K, K)], sem_st[b])

    def outer(t, carry):
        for u in range(NB):
            j = t * NB + u

            @pl.when(j < CH)
            def _():
                iter_j(j, u)

        return carry

    lax.fori_loop(0, -(-CH // NB), outer, 0)
    wait_store((CH - 2) % NB)
    wait_store((CH - 1) % NB)


def _sc_scatter_body(eupd_hbm, dst_hbm, zeros_hbm, out_hbm,
                     idxd, rows, agg_sp,
                     sr0, sr1, sr2, sr3, ss0, ss1, ss2, ss3):
    sem_r = (sr0, sr1, sr2, sr3)
    sem_s = (ss0, ss1, ss2, ss3)
    cid = lax.axis_index("c")
    sid = lax.axis_index("s")
    wid = cid * NS + sid
    w0 = wid * WE

    pltpu.sync_copy(zeros_hbm.at[pl.ds(sid * RPT, RPT)],
                    agg_sp.at[pl.ds(sid * RPT, RPT)])
    plsc.subcore_barrier()

    def fire(j, b):
        pltpu.async_copy(dst_hbm.at[pl.ds(w0 + j * K, K)], idxd.at[b], sem_r[b])
        pltpu.async_copy(eupd_hbm.at[pl.ds(w0 + j * K, K)], rows.at[b], sem_r[b])

    def wait_rows(b):
        pltpu.make_async_copy(dst_hbm.at[pl.ds(w0, K)], idxd.at[b], sem_r[b]).wait()
        pltpu.make_async_copy(eupd_hbm.at[pl.ds(w0, K)], rows.at[b], sem_r[b]).wait()

    def wait_scatter(b):
        pltpu.make_async_copy(rows.at[b], agg_sp.at[pl.ds(0, K)], sem_s[b]).wait()

    fire(0, 0)
    fire(1, 1)

    def iter_j(j, b):
        bn = (b + 2) % NB

        @pl.when(j >= 2)
        def _():
            wait_scatter(bn)

        @pl.when(j + 2 < CH)
        def _():
            fire(j + 2, bn)

        wait_rows(b)
        pltpu.async_copy(rows.at[b], agg_sp.at[idxd.at[b]], sem_s[b], add=True)

    def outer(t, carry):
        for u in range(NB):
            j = t * NB + u

            @pl.when(j < CH)
            def _():
                iter_j(j, u)

        return carry

    lax.fori_loop(0, -(-CH // NB), outer, 0)
    wait_scatter((CH - 2) % NB)
    wait_scatter((CH - 1) % NB)
    plsc.subcore_barrier()
    pltpu.sync_copy(agg_sp.at[pl.ds(sid * RPT, RPT)],
                    out_hbm.at[cid, pl.ds(sid * RPT, RPT)])


@functools.lru_cache(maxsize=None)
def _sc_kernels():
    mesh = plsc.VectorSubcoreMesh(
        core_axis_name="c", subcore_axis_name="s",
        num_cores=NC, num_subcores=NS,
    )
    gather = pl.kernel(
        _sc_gather_body,
        out_type=jax.ShapeDtypeStruct((E, D), jnp.float32),
        mesh=mesh,
        scratch_types=(
            [
                pltpu.VMEM((NB, K), jnp.int32),
                pltpu.VMEM((NB, K), jnp.int32),
                pltpu.VMEM((NB, K, D), jnp.float32),
                pltpu.VMEM((NB, K, D), jnp.float32),
            ]
            + [pltpu.SemaphoreType.DMA] * 12
        ),
    )
    scatter = pl.kernel(
        _sc_scatter_body,
        out_type=jax.ShapeDtypeStruct((NC, NP, D), jnp.float32),
        mesh=mesh,
        scratch_types=(
            [
                pltpu.VMEM((NB, K), jnp.int32),
                pltpu.VMEM((NB, K, D), jnp.float32),
                pltpu.VMEM_SHARED((NP, D), jnp.float32),
            ]
            + [pltpu.SemaphoreType.DMA] * 8
        ),
    )
    return gather, scatter


def _proj_body(v_ref, wa_ref, wb_ref, p_ref, q_ref):
    v = v_ref[...]
    p_ref[...] = lax.dot_general(v, wa_ref[...], (((1,), (0,)), ((), ())),
                                 precision=_HI)
    q_ref[...] = lax.dot_general(v, wb_ref[...], (((1,), (0,)), ((), ())),
                                 precision=_HI)


def _edge_body(s_ref, e_ref, wc_ref, be_ref, enew_ref, eupd_ref):
    e = e_ref[...]
    acc = (s_ref[...]
           + lax.dot_general(e, wc_ref[...], (((1,), (0,)), ((), ())))
           + be_ref[...])
    upd = jnp.maximum(acc, 0.0)
    eupd_ref[...] = upd
    enew_ref[...] = e + upd


def _vtx_body(v_ref, a0_ref, a1_ref, wva_ref, wvb_ref, bv_ref, out_ref):
    v = v_ref[...]
    agg = a0_ref[...] + a1_ref[...]
    h = (lax.dot_general(v, wva_ref[...], (((1,), (0,)), ((), ())),
                         precision=_HI)
         + lax.dot_general(agg, wvb_ref[...], (((1,), (0,)), ((), ())),
                           precision=_HI)
         + bv_ref[...])
    out_ref[...] = v + jnp.maximum(h, 0.0)


_VB = 2000
_EB = 8000


def _proj(v, wa, wb):
    return pl.pallas_call(
        _proj_body,
        grid=(N // _VB,),
        in_specs=[
            pl.BlockSpec((_VB, D), lambda i: (i, 0)),
            pl.BlockSpec((D, D), lambda i: (0, 0)),
            pl.BlockSpec((D, D), lambda i: (0, 0)),
        ],
        out_specs=[
            pl.BlockSpec((_VB, D), lambda i: (i, 0)),
            pl.BlockSpec((_VB, D), lambda i: (i, 0)),
        ],
        out_shape=[
            jax.ShapeDtypeStruct((N, D), jnp.float32),
            jax.ShapeDtypeStruct((N, D), jnp.float32),
        ],
    )(v, wa, wb)


def _edge(s, e, wc, be2d):
    return pl.pallas_call(
        _edge_body,
        grid=(E // _EB,),
        in_specs=[
            pl.BlockSpec((_EB, D), lambda i: (i, 0)),
            pl.BlockSpec((_EB, D), lambda i: (i, 0)),
            pl.BlockSpec((D, D), lambda i: (0, 0)),
            pl.BlockSpec((1, D), lambda i: (0, 0)),
        ],
        out_specs=[
            pl.BlockSpec((_EB, D), lambda i: (i, 0)),
            pl.BlockSpec((_EB, D), lambda i: (i, 0)),
        ],
        out_shape=[
            jax.ShapeDtypeStruct((E, D), jnp.float32),
            jax.ShapeDtypeStruct((E, D), jnp.float32),
        ],
    )(s, e, wc, be2d)


def _vtx(v, a0, a1, wva, wvb, bv2d):
    return pl.pallas_call(
        _vtx_body,
        grid=(N // _VB,),
        in_specs=[
            pl.BlockSpec((_VB, D), lambda i: (i, 0)),
            pl.BlockSpec((_VB, D), lambda i: (i, 0)),
            pl.BlockSpec((_VB, D), lambda i: (i, 0)),
            pl.BlockSpec((D, D), lambda i: (0, 0)),
            pl.BlockSpec((D, D), lambda i: (0, 0)),
            pl.BlockSpec((1, D), lambda i: (0, 0)),
        ],
        out_specs=pl.BlockSpec((_VB, D), lambda i: (i, 0)),
        out_shape=jax.ShapeDtypeStruct((N, D), jnp.float32),
    )(v, a0, a1, wva, wvb, bv2d)


def kernel(vertex_features, edge_features, edge_index,
           We0, be0, Wv0, bv0, We1, be1, Wv1, bv1):
    src = edge_index[0]
    dst = edge_index[1]
    zeros = jnp.zeros((NP, D), jnp.float32)

    sc_gather, sc_scatter = _sc_kernels()
    v, e = vertex_features, edge_features
    for (We, be, Wv, bv) in ((We0, be0, Wv0, bv0), (We1, be1, Wv1, bv1)):
        p, q = _proj(v, We[:D], We[D:2 * D])
        s = sc_gather(p, q, src, dst)
        e, eupd = _edge(s, e, We[2 * D:], be.reshape(1, D))
        aggp = sc_scatter(eupd, dst, zeros)
        v = _vtx(v, aggp[0], aggp[1], Wv[:D], Wv[D:], bv.reshape(1, D))
    return v, e
